# baseline (speedup 1.0000x reference)
import functools

import jax
import jax.numpy as jnp
from jax import lax
from jax.experimental import pallas as pl
from jax.experimental.pallas import tpu as pltpu
from jax.experimental.pallas import tpu_sc as plsc

_N = 10000
_K = 32
_NH = 128
_SCALE = 30.0
_EPS = 1e-5

_BN = 400
_BNK = _BN * _K
_GRID = _N // _BN

_PARTS = ((0, 5200), (5200, 4800))
_BN_N = 400
_BNK_N = _BN_N * _K

_NW = 32


def _gelu(x):
    return x * 0.5 * (1.0 + lax.erf(x * (2.0 ** -0.5)))


def _ln(x, g, b):
    m = jnp.mean(x, axis=-1, keepdims=True)
    v = jnp.mean((x - m) ** 2, axis=-1, keepdims=True)
    return (x - m) / jnp.sqrt(v + _EPS) * g + b


def _dot(a, b):
    return jnp.dot(a, b, preferred_element_type=jnp.float32)


def _pre_body(hv_ref, wv_ref, wg_ref, b1_ref, s1_ref, p1_ref):
    hv = hv_ref[...]
    s1_ref[...] = _dot(hv, wv_ref[...]) + b1_ref[...]
    p1_ref[...] = _dot(hv, wg_ref[...])


def _pre(hv, wv, wg, b1):
    return pl.pallas_call(
        _pre_body,
        out_shape=(
            jax.ShapeDtypeStruct((_N, _NH), jnp.float32),
            jax.ShapeDtypeStruct((_N, _NH), jnp.float32),
        ),
    )(hv, wv, wg, b1)


def _make_gather(offset, nrows):
    per_w = nrows // _NW
    chunk, nbuf, pd = (80, 8, 4) if per_w % 80 == 0 else (40, 8, 4)
    nchunk = per_w // chunk
    ngrp = (nchunk - pd - nbuf) // nbuf + 1
    mesh = plsc.VectorSubcoreMesh(core_axis_name="c", subcore_axis_name="s",
                                  num_cores=2, num_subcores=16)

    @functools.partial(
        pl.kernel,
        out_type=jax.ShapeDtypeStruct((nrows, _NH), jnp.float32),
        mesh=mesh,
        scratch_types=[
            pltpu.VMEM((per_w,), jnp.int32),
            pltpu.VMEM((nbuf, chunk, _NH), jnp.float32),
            [pltpu.SemaphoreType.DMA] * nbuf,
            [pltpu.SemaphoreType.DMA] * nbuf,
        ],
    )
    def gk(table_hbm, idx_hbm, out_hbm, idx_v, rows_v, gsem, ssem):
        wid = lax.axis_index("s") * 2 + lax.axis_index("c")
        base = wid * per_w
        pltpu.sync_copy(idx_hbm.at[pl.ds(offset + base, per_w)], idx_v)

        def idx_win(c):
            return idx_v.at[pl.ds(pl.multiple_of(c * chunk, 8), chunk)]

        def start_gather(c, b):
            pltpu.async_copy(table_hbm.at[idx_win(c)], rows_v.at[b], gsem[b])

        def wait_gather(c, b):
            pltpu.make_async_copy(table_hbm.at[idx_win(c)], rows_v.at[b],
                                  gsem[b]).wait()

        def start_store(c, b):
            pltpu.async_copy(rows_v.at[b],
                             out_hbm.at[pl.ds(base + c * chunk, chunk)],
                             ssem[b])

        def wait_store(c, b):
            pltpu.make_async_copy(rows_v.at[b],
                                  out_hbm.at[pl.ds(base + c * chunk, chunk)],
                                  ssem[b]).wait()

        def step(c, b, wait_st, prefetch):
            wait_gather(c, b)
            start_store(c, b)
            if prefetch:
                nb = (b + pd) % nbuf
                if wait_st:
                    wait_store(c - pd, nb)
                start_gather(c + pd, nb)

        for c in range(pd):
            start_gather(c, c)
        for c in range(pd):
            step(c, c, wait_st=False, prefetch=True)
        for c in range(pd, nbuf):
            step(c, c, wait_st=True, prefetch=True)

        def group(g, carry):
            c0 = g * nbuf
            for b in range(nbuf):
                step(c0 + b, b, wait_st=True, prefetch=True)
            return carry

        lax.fori_loop(1, ngrp, group, 0)

        for c in range(ngrp * nbuf, nchunk):
            step(c, c % nbuf, wait_st=True, prefetch=(c + pd < nchunk))
        for c in range(nchunk - nbuf, nchunk):
            wait_store(c, c % nbuf)

    return gk


def _node_body(he_ref, g_ref, hv_ref, s1_ref,
               we_ref, w2_ref, b2_ref, w3_ref, b3_ref,
               win_ref, bin_ref, wout_ref, bout_ref,
               g1_ref, bt1_ref, g2_ref, bt2_ref,
               w11v_ref, w11g_ref, b11_ref,
               hv2_ref, p2_ref, s2_ref):
    x = _dot(he_ref[...], we_ref[...]) + g_ref[...]
    x = x.reshape(_BN_N, _K, _NH) + s1_ref[...][:, None, :]
    x = _gelu(x).reshape(_BNK_N, _NH)
    x = _gelu(_dot(x, w2_ref[...]) + b2_ref[...])
    m = _dot(x, w3_ref[...]) + b3_ref[...]
    dh = m.reshape(_BN_N, _K, _NH).sum(axis=1) * (1.0 / _SCALE)
    h = _ln(hv_ref[...] + dh, g1_ref[...], bt1_ref[...])
    ff = _dot(_gelu(_dot(h, win_ref[...]) + bin_ref[...]), wout_ref[...]) + bout_ref[...]
    h2 = _ln(h + ff, g2_ref[...], bt2_ref[...])
    hv2_ref[...] = h2
    p2_ref[...] = _dot(h2, w11g_ref[...])
    s2_ref[...] = _dot(h2, w11v_ref[...]) + b11_ref[...]


def _node(node_off, pn, he2d, grows, hv, s1, we, w2, b2, w3, b3, win, bin_,
          wout, bout, g1, bt1, g2, bt2, w11v, w11g, b11):
    off = node_off // _BN_N
    hgrid = pn // _BN_N
    full = lambda shape: pl.BlockSpec(shape, lambda i: (0, 0))
    return pl.pallas_call(
        _node_body,
        grid=(hgrid,),
        in_specs=[
            pl.BlockSpec((_BNK_N, _NH), lambda i: (i + off, 0)),
            pl.BlockSpec((_BNK_N, _NH), lambda i: (i, 0)),
            pl.BlockSpec((_BN_N, _NH), lambda i: (i + off, 0)),
            pl.BlockSpec((_BN_N, _NH), lambda i: (i + off, 0)),
            full((_NH, _NH)), full((_NH, _NH)), full((1, _NH)),
            full((_NH, _NH)), full((1, _NH)),
            full((_NH, 4 * _NH)), full((1, 4 * _NH)),
            full((4 * _NH, _NH)), full((1, _NH)),
            full((1, _NH)), full((1, _NH)), full((1, _NH)), full((1, _NH)),
            full((_NH, _NH)), full((_NH, _NH)), full((1, _NH)),
        ],
        out_specs=(
            pl.BlockSpec((_BN_N, _NH), lambda i: (i, 0)),
            pl.BlockSpec((_BN_N, _NH), lambda i: (i, 0)),
            pl.BlockSpec((_BN_N, _NH), lambda i: (i, 0)),
        ),
        out_shape=(
            jax.ShapeDtypeStruct((pn, _NH), jnp.float32),
            jax.ShapeDtypeStruct((pn, _NH), jnp.float32),
            jax.ShapeDtypeStruct((pn, _NH), jnp.float32),
        ),
        compiler_params=pltpu.CompilerParams(
            dimension_semantics=("arbitrary",),
        ),
    )(he2d, grows, hv, s1, we, w2, b2, w3, b3, win, bin_, wout, bout,
      g1, bt1, g2, bt2, w11v, w11g, b11)


def _edge_body(he_ref, g_ref, s2_ref,
               we_ref, w12_ref, b12_ref, w13_ref, b13_ref,
               g3_ref, bt3_ref, out_ref):
    he = he_ref[...]
    x = _dot(he, we_ref[...]) + g_ref[...]
    x = x.reshape(_BN, _K, _NH) + s2_ref[...][:, None, :]
    x = _gelu(x).reshape(_BNK, _NH)
    x = _gelu(_dot(x, w12_ref[...]) + b12_ref[...])
    m = _dot(x, w13_ref[...]) + b13_ref[...]
    out_ref[...] = _ln(he + m, g3_ref[...], bt3_ref[...])


def _edge(he2d, grows, s2, we, w12, b12, w13, b13, g3, bt3):
    full = lambda shape: pl.BlockSpec(shape, lambda i: (0, 0))
    return pl.pallas_call(
        _edge_body,
        grid=(_GRID,),
        in_specs=[
            pl.BlockSpec((_BNK, _NH), lambda i: (i, 0)),
            pl.BlockSpec((_BNK, _NH), lambda i: (i, 0)),
            pl.BlockSpec((_BN, _NH), lambda i: (i, 0)),
            full((_NH, _NH)), full((_NH, _NH)), full((1, _NH)),
            full((_NH, _NH)), full((1, _NH)),
            full((1, _NH)), full((1, _NH)),
        ],
        out_specs=pl.BlockSpec((_BNK, _NH), lambda i: (i, 0)),
        out_shape=jax.ShapeDtypeStruct((_N * _K, _NH), jnp.float32),
        compiler_params=pltpu.CompilerParams(
            dimension_semantics=("arbitrary",),
        ),
    )(he2d, grows, s2, we, w12, b12, w13, b13, g3, bt3)


def kernel(h_V, h_E, E_idx, W1, b1, W2, b2, W3, b3, Win, bin_, Wout, bout,
           W11, b11, W12, b12, W13, b13, g1, bt1, g2, bt2, g3, bt3):
    hv = h_V[0]
    he2d = h_E.reshape(_N * _K, _NH)
    idx = E_idx.reshape(_N * _K).astype(jnp.int32)

    r = lambda v: v.reshape(1, -1)
    w1v, w1e, w1g = W1[:_NH], W1[_NH:2 * _NH], W1[2 * _NH:]
    w11v, w11e, w11g = W11[:_NH], W11[_NH:2 * _NH], W11[2 * _NH:]

    s1, p1 = _pre(hv, w1v, w1g, r(b1))

    parts = []
    for node_off, pn in _PARTS:
        grows = _make_gather(node_off * _K, pn * _K)(p1, idx)
        parts.append(_node(
            node_off, pn, he2d, grows, hv, s1, w1e, W2, r(b2), W3, r(b3),
            Win, r(bin_), Wout, r(bout), r(g1), r(bt1), r(g2), r(bt2),
            w11v, w11g, r(b11)))
    hv2 = jnp.concatenate([q[0] for q in parts], axis=0)
    p2 = jnp.concatenate([q[1] for q in parts], axis=0)
    s2 = jnp.concatenate([q[2] for q in parts], axis=0)

    gather_full = _make_gather(0, _N * _K)
    grows2 = gather_full(p2, idx)
    he_out = _edge(he2d, grows2, s2, w11e, W12, r(b12), W13, r(b13),
                   r(g3), r(bt3))
    return hv2[None], he_out.reshape(1, _N, _K, _NH)


# speedup vs baseline: 1.0140x; 1.0140x over previous
import functools

import jax
import jax.numpy as jnp
from jax import lax
from jax.experimental import pallas as pl
from jax.experimental.pallas import tpu as pltpu
from jax.experimental.pallas import tpu_sc as plsc

_N = 10000
_K = 32
_NH = 128
_SCALE = 30.0
_EPS = 1e-5

_BN = 400
_BNK = _BN * _K
_GRID = _N // _BN

_PARTS = ((0, _N),)
_BN_N = 400
_BNK_N = _BN_N * _K

_NW = 32


def _gelu(x):
    return x * 0.5 * (1.0 + lax.erf(x * (2.0 ** -0.5)))


def _ln(x, g, b):
    m = jnp.mean(x, axis=-1, keepdims=True)
    v = jnp.mean((x - m) ** 2, axis=-1, keepdims=True)
    return (x - m) / jnp.sqrt(v + _EPS) * g + b


def _dot(a, b):
    return jnp.dot(a, b, preferred_element_type=jnp.float32)


def _pre_body(hv_ref, wv_ref, wg_ref, b1_ref, s1_ref, p1_ref):
    hv = hv_ref[...]
    s1_ref[...] = _dot(hv, wv_ref[...]) + b1_ref[...]
    p1_ref[...] = _dot(hv, wg_ref[...])


def _pre(hv, wv, wg, b1):
    return pl.pallas_call(
        _pre_body,
        out_shape=(
            jax.ShapeDtypeStruct((_N, _NH), jnp.float32),
            jax.ShapeDtypeStruct((_N, _NH), jnp.float32),
        ),
    )(hv, wv, wg, b1)


def _make_gather(offset, nrows):
    per_w = nrows // _NW
    chunk, nbuf, pd = (80, 8, 4) if per_w % 80 == 0 else (40, 8, 4)
    nchunk = per_w // chunk
    ngrp = (nchunk - pd - nbuf) // nbuf + 1
    mesh = plsc.VectorSubcoreMesh(core_axis_name="c", subcore_axis_name="s",
                                  num_cores=2, num_subcores=16)

    @functools.partial(
        pl.kernel,
        out_type=jax.ShapeDtypeStruct((nrows, _NH), jnp.float32),
        mesh=mesh,
        scratch_types=[
            pltpu.VMEM((per_w,), jnp.int32),
            pltpu.VMEM((nbuf, chunk, _NH), jnp.float32),
            [pltpu.SemaphoreType.DMA] * nbuf,
            [pltpu.SemaphoreType.DMA] * nbuf,
        ],
    )
    def gk(table_hbm, idx_hbm, out_hbm, idx_v, rows_v, gsem, ssem):
        wid = lax.axis_index("s") * 2 + lax.axis_index("c")
        base = wid * per_w
        pltpu.sync_copy(idx_hbm.at[pl.ds(offset + base, per_w)], idx_v)

        def idx_win(c):
            return idx_v.at[pl.ds(pl.multiple_of(c * chunk, 8), chunk)]

        def start_gather(c, b):
            pltpu.async_copy(table_hbm.at[idx_win(c)], rows_v.at[b], gsem[b])

        def wait_gather(c, b):
            pltpu.make_async_copy(table_hbm.at[idx_win(c)], rows_v.at[b],
                                  gsem[b]).wait()

        def start_store(c, b):
            pltpu.async_copy(rows_v.at[b],
                             out_hbm.at[pl.ds(base + c * chunk, chunk)],
                             ssem[b])

        def wait_store(c, b):
            pltpu.make_async_copy(rows_v.at[b],
                                  out_hbm.at[pl.ds(base + c * chunk, chunk)],
                                  ssem[b]).wait()

        def step(c, b, wait_st, prefetch):
            wait_gather(c, b)
            start_store(c, b)
            if prefetch:
                nb = (b + pd) % nbuf
                if wait_st:
                    wait_store(c - pd, nb)
                start_gather(c + pd, nb)

        for c in range(pd):
            start_gather(c, c)
        for c in range(pd):
            step(c, c, wait_st=False, prefetch=True)
        for c in range(pd, nbuf):
            step(c, c, wait_st=True, prefetch=True)

        def group(g, carry):
            c0 = g * nbuf
            for b in range(nbuf):
                step(c0 + b, b, wait_st=True, prefetch=True)
            return carry

        lax.fori_loop(1, ngrp, group, 0)

        for c in range(ngrp * nbuf, nchunk):
            step(c, c % nbuf, wait_st=True, prefetch=(c + pd < nchunk))
        for c in range(nchunk - nbuf, nchunk):
            wait_store(c, c % nbuf)

    return gk


def _node_body(he_ref, g_ref, hv_ref, s1_ref,
               we_ref, w2_ref, b2_ref, w3_ref, b3_ref,
               win_ref, bin_ref, wout_ref, bout_ref,
               g1_ref, bt1_ref, g2_ref, bt2_ref,
               w11v_ref, w11g_ref, b11_ref,
               hv2_ref, p2_ref, s2_ref):
    x = _dot(he_ref[...], we_ref[...]) + g_ref[...]
    x = x.reshape(_BN_N, _K, _NH) + s1_ref[...][:, None, :]
    x = _gelu(x).reshape(_BNK_N, _NH)
    x = _gelu(_dot(x, w2_ref[...]) + b2_ref[...])
    m = _dot(x, w3_ref[...]) + b3_ref[...]
    dh = m.reshape(_BN_N, _K, _NH).sum(axis=1) * (1.0 / _SCALE)
    h = _ln(hv_ref[...] + dh, g1_ref[...], bt1_ref[...])
    ff = _dot(_gelu(_dot(h, win_ref[...]) + bin_ref[...]), wout_ref[...]) + bout_ref[...]
    h2 = _ln(h + ff, g2_ref[...], bt2_ref[...])
    hv2_ref[...] = h2
    p2_ref[...] = _dot(h2, w11g_ref[...])
    s2_ref[...] = _dot(h2, w11v_ref[...]) + b11_ref[...]


def _node(node_off, pn, he2d, grows, hv, s1, we, w2, b2, w3, b3, win, bin_,
          wout, bout, g1, bt1, g2, bt2, w11v, w11g, b11):
    off = node_off // _BN_N
    hgrid = pn // _BN_N
    full = lambda shape: pl.BlockSpec(shape, lambda i: (0, 0))
    return pl.pallas_call(
        _node_body,
        grid=(hgrid,),
        in_specs=[
            pl.BlockSpec((_BNK_N, _NH), lambda i: (i + off, 0)),
            pl.BlockSpec((_BNK_N, _NH), lambda i: (i, 0)),
            pl.BlockSpec((_BN_N, _NH), lambda i: (i + off, 0)),
            pl.BlockSpec((_BN_N, _NH), lambda i: (i + off, 0)),
            full((_NH, _NH)), full((_NH, _NH)), full((1, _NH)),
            full((_NH, _NH)), full((1, _NH)),
            full((_NH, 4 * _NH)), full((1, 4 * _NH)),
            full((4 * _NH, _NH)), full((1, _NH)),
            full((1, _NH)), full((1, _NH)), full((1, _NH)), full((1, _NH)),
            full((_NH, _NH)), full((_NH, _NH)), full((1, _NH)),
        ],
        out_specs=(
            pl.BlockSpec((_BN_N, _NH), lambda i: (i, 0)),
            pl.BlockSpec((_BN_N, _NH), lambda i: (i, 0)),
            pl.BlockSpec((_BN_N, _NH), lambda i: (i, 0)),
        ),
        out_shape=(
            jax.ShapeDtypeStruct((pn, _NH), jnp.float32),
            jax.ShapeDtypeStruct((pn, _NH), jnp.float32),
            jax.ShapeDtypeStruct((pn, _NH), jnp.float32),
        ),
        compiler_params=pltpu.CompilerParams(
            dimension_semantics=("arbitrary",),
        ),
    )(he2d, grows, hv, s1, we, w2, b2, w3, b3, win, bin_, wout, bout,
      g1, bt1, g2, bt2, w11v, w11g, b11)


def _edge_body(he_ref, g_ref, s2_ref,
               we_ref, w12_ref, b12_ref, w13_ref, b13_ref,
               g3_ref, bt3_ref, out_ref):
    he = he_ref[...]
    x = _dot(he, we_ref[...]) + g_ref[...]
    x = x.reshape(_BN, _K, _NH) + s2_ref[...][:, None, :]
    x = _gelu(x).reshape(_BNK, _NH)
    x = _gelu(_dot(x, w12_ref[...]) + b12_ref[...])
    m = _dot(x, w13_ref[...]) + b13_ref[...]
    out_ref[...] = _ln(he + m, g3_ref[...], bt3_ref[...])


def _edge(he2d, grows, s2, we, w12, b12, w13, b13, g3, bt3):
    full = lambda shape: pl.BlockSpec(shape, lambda i: (0, 0))
    return pl.pallas_call(
        _edge_body,
        grid=(_GRID,),
        in_specs=[
            pl.BlockSpec((_BNK, _NH), lambda i: (i, 0)),
            pl.BlockSpec((_BNK, _NH), lambda i: (i, 0)),
            pl.BlockSpec((_BN, _NH), lambda i: (i, 0)),
            full((_NH, _NH)), full((_NH, _NH)), full((1, _NH)),
            full((_NH, _NH)), full((1, _NH)),
            full((1, _NH)), full((1, _NH)),
        ],
        out_specs=pl.BlockSpec((_BNK, _NH), lambda i: (i, 0)),
        out_shape=jax.ShapeDtypeStruct((_N * _K, _NH), jnp.float32),
        compiler_params=pltpu.CompilerParams(
            dimension_semantics=("arbitrary",),
        ),
    )(he2d, grows, s2, we, w12, b12, w13, b13, g3, bt3)


def kernel(h_V, h_E, E_idx, W1, b1, W2, b2, W3, b3, Win, bin_, Wout, bout,
           W11, b11, W12, b12, W13, b13, g1, bt1, g2, bt2, g3, bt3):
    hv = h_V[0]
    he2d = h_E.reshape(_N * _K, _NH)
    idx = E_idx.reshape(_N * _K).astype(jnp.int32)

    r = lambda v: v.reshape(1, -1)
    w1v, w1e, w1g = W1[:_NH], W1[_NH:2 * _NH], W1[2 * _NH:]
    w11v, w11e, w11g = W11[:_NH], W11[_NH:2 * _NH], W11[2 * _NH:]

    s1, p1 = _pre(hv, w1v, w1g, r(b1))

    parts = []
    for node_off, pn in _PARTS:
        grows = _make_gather(node_off * _K, pn * _K)(p1, idx)
        parts.append(_node(
            node_off, pn, he2d, grows, hv, s1, w1e, W2, r(b2), W3, r(b3),
            Win, r(bin_), Wout, r(bout), r(g1), r(bt1), r(g2), r(bt2),
            w11v, w11g, r(b11)))
    hv2 = jnp.concatenate([q[0] for q in parts], axis=0)
    p2 = jnp.concatenate([q[1] for q in parts], axis=0)
    s2 = jnp.concatenate([q[2] for q in parts], axis=0)

    gather_full = _make_gather(0, _N * _K)
    grows2 = gather_full(p2, idx)
    he_out = _edge(he2d, grows2, s2, w11e, W12, r(b12), W13, r(b13),
                   r(g3), r(bt3))
    return hv2[None], he_out.reshape(1, _N, _K, _NH)
